# Initial kernel scaffold; baseline (speedup 1.0000x reference)
import jax
import jax.numpy as jnp
from jax.experimental import pallas as pl


def kernel(values, offsets, M):
    raise NotImplementedError("write your pallas kernel here")


# speedup vs baseline: 2.7433x; 2.7433x over previous
import functools

import jax
import jax.numpy as jnp
from jax import lax
from jax.experimental import pallas as pl
from jax.experimental.pallas import tpu as pltpu
from jax.experimental.pallas import tpu_sc as plsc

_EPS = 1e-6
_L = 16


def _rsqrt_newton(x):
    i = plsc.bitcast(x, jnp.int32)
    i = jnp.full(x.shape, 0x5F3759DF, jnp.int32) - lax.shift_right_logical(i, 1)
    y = plsc.bitcast(i, jnp.float32)
    for _ in range(3):
        y = y * (1.5 - 0.5 * x * y * y)
    return y


@functools.lru_cache(maxsize=None)
def _build(total, M, B):
    mesh = plsc.VectorSubcoreMesh(core_axis_name="c", subcore_axis_name="s")
    NC, NS = mesh.num_cores, mesh.num_subcores
    NW = NC * NS
    R = total // NW
    NT = 2
    SUB = R // NT
    CV = M // _L
    assert total == NW * NT * SUB and M % _L == 0

    def seg_bounds(off_vec, i, base):
        oa = off_vec[i]
        ob = jnp.int32(total) if i == B - 1 else off_vec[i + 1]
        a = jnp.clip(oa - base, 0, SUB)
        b = jnp.clip(ob - base, 0, SUB)
        return a, b

    @functools.partial(
        pl.kernel,
        out_type=jax.ShapeDtypeStruct((NW, 2 * _L), jnp.float32),
        mesh=mesh,
        compiler_params=pltpu.CompilerParams(needs_layout_passes=False),
        scratch_types=[
            pltpu.VMEM((SUB, M), jnp.float32),
            pltpu.VMEM((_L,), jnp.int32),
            pltpu.VMEM((2 * _L,), jnp.float32),
        ],
    )
    def stats_k(values_hbm, offsets_hbm, part_hbm, chunk, offs, stat_v):
        zeros = jnp.zeros((_L,), jnp.float32)
        lane_iota = lax.iota(jnp.int32, _L)
        wid = lax.axis_index("c") * NS + lax.axis_index("s")
        lo = wid * R
        pltpu.sync_copy(offsets_hbm.at[pl.ds(0, _L)], offs)
        off_vec = offs[...]

        sums_vec = zeros
        sq_vec = zeros
        for t in range(NT):
            base = lo + t * SUB
            pltpu.sync_copy(values_hbm.at[pl.ds(base, SUB)], chunk)
            for i in range(B):
                a, b = seg_bounds(off_vec, i, base)

                def body(r, carry):
                    s, q = carry
                    for cc in range(CV):
                        v = chunk[r, pl.ds(cc * _L, _L)]
                        s = s + v
                        q = q + v * v
                    return s, q

                s, q = lax.fori_loop(a, b, body, (zeros, zeros))
                lane = lane_iota == i
                sums_vec = jnp.where(lane, sums_vec + jnp.sum(s), sums_vec)
                sq_vec = jnp.where(lane, sq_vec + jnp.sum(q), sq_vec)

        stat_v[pl.ds(0, _L)] = sums_vec
        stat_v[pl.ds(_L, _L)] = sq_vec
        pltpu.sync_copy(stat_v, part_hbm.at[wid])

    @functools.partial(
        pl.kernel,
        out_type=jax.ShapeDtypeStruct((total, M), jnp.float32),
        mesh=mesh,
        compiler_params=pltpu.CompilerParams(needs_layout_passes=False),
        scratch_types=[
            pltpu.VMEM((SUB, M), jnp.float32),
            pltpu.VMEM((_L,), jnp.int32),
            pltpu.VMEM((NW, 2 * _L), jnp.float32),
        ],
    )
    def norm_k(values_hbm, offsets_hbm, part_hbm, out_hbm,
               chunk, offs, part_v):
        zeros = jnp.zeros((_L,), jnp.float32)
        lane_iota = lax.iota(jnp.int32, _L)
        wid = lax.axis_index("c") * NS + lax.axis_index("s")
        lo = wid * R
        pltpu.sync_copy(offsets_hbm.at[pl.ds(0, _L)], offs)
        pltpu.sync_copy(part_hbm, part_v)
        off_vec = offs[...]

        sums = zeros
        sqs = zeros
        for w in range(NW):
            sums = sums + part_v[w, pl.ds(0, _L)]
            sqs = sqs + part_v[w, pl.ds(_L, _L)]

        off_hi = jnp.full((_L,), total, jnp.int32)
        for i in range(B - 1):
            off_hi = jnp.where(lane_iota == i, off_vec[i + 1], off_hi)
        n_elem = (off_hi - off_vec).astype(jnp.float32) * jnp.float32(M)

        mean = sums / n_elem
        var = sqs / n_elem - mean * mean
        rstd = _rsqrt_newton(var + _EPS)

        for t in range(NT):
            base = lo + t * SUB
            pltpu.sync_copy(values_hbm.at[pl.ds(base, SUB)], chunk)
            for i in range(B):
                a, b = seg_bounds(off_vec, i, base)
                mv = jnp.broadcast_to(mean[i], (_L,))
                rv = jnp.broadcast_to(rstd[i], (_L,))

                def body(r, carry):
                    for cc in range(CV):
                        v = chunk[r, pl.ds(cc * _L, _L)]
                        chunk[r, pl.ds(cc * _L, _L)] = (v - mv) * rv
                    return carry

                lax.fori_loop(a, b, body, 0)

            pltpu.sync_copy(chunk, out_hbm.at[pl.ds(base, SUB)])

    return stats_k, norm_k


def kernel(values, offsets, M):
    total, m = values.shape
    B = offsets.shape[0] - 1
    stats_k, norm_k = _build(total, m, B)
    part = stats_k(values, offsets)
    return norm_k(values, offsets, part)
